# baseline (speedup 1.0000x reference)
import jax
import jax.numpy as jnp
from jax import lax
from jax.experimental import pallas as pl
from jax.experimental.pallas import tpu as pltpu
from jax.experimental.pallas import tpu_sc as plsc

L1 = 3072
NPSQT = 8
D = L1 + NPSQT
DW = D // 2
DWP = 1552
LW = L1 // 2
H = L1 // 2
B = 1024
A = 32
NC = 2
NS = 16
NW = NC * NS
SPW = B // NW
ROWS_PER_GATHER = 16
ACC = L1 + 32


def _sc_kernel(w_idx, w_val, b_idx, b_val, us, them, weight, bias, ftv, out,
               idxw_v, idxb_v, vw_v, vb_v, us_v, them_v, ft_v, bias_v,
               rows0, rows1, acc_w, acc_b, out_stage, sem0, sem1):
    wid = lax.axis_index("s") * NC + lax.axis_index("c")
    base = wid * SPW

    pltpu.sync_copy(w_idx.at[pl.ds(base, SPW)], idxw_v)
    pltpu.sync_copy(b_idx.at[pl.ds(base, SPW)], idxb_v)
    pltpu.sync_copy(w_val.at[pl.ds(base, SPW)], vw_v)
    pltpu.sync_copy(b_val.at[pl.ds(base, SPW)], vb_v)
    pltpu.sync_copy(us.at[pl.ds(base, SPW)], us_v)
    pltpu.sync_copy(them.at[pl.ds(base, SPW)], them_v)
    pltpu.sync_copy(bias, bias_v)
    pltpu.sync_copy(ftv, ft_v)

    lane = lax.broadcasted_iota(jnp.int32, (16,), 0)
    tail_mask = (lane >= 12).astype(jnp.float32)
    himask = jnp.full((16,), jnp.int32(-65536))
    rows = (rows0, rows1)
    sems = (sem0, sem1)
    idxs = (idxw_v, idxb_v)
    vals = (vw_v, vb_v)

    def lo_f32(x):
        return plsc.bitcast(jnp.left_shift(x, 16), jnp.float32)

    def hi_f32(x):
        return plsc.bitcast(jnp.bitwise_and(x, himask), jnp.float32)

    def start_gather(s, p, h):
        src = weight.at[idxs[p].at[s, pl.ds(h * ROWS_PER_GATHER, ROWS_PER_GATHER)]]
        return pltpu.async_copy(src, rows[h], sems[h])

    def full16(v):
        return jnp.full((16,), v, dtype=jnp.int32)

    G = 8
    GW = G // 2
    NG = LW // (GW * 16)

    def mac(s, p, h):
        acc = (acc_w, acc_b)[p]
        rbuf = rows[h]
        vref = vals[p]
        first = (h == 0)

        vjs = [plsc.load_gather(vref, [full16(s), full16(h * ROWS_PER_GATHER + j)])
               for j in range(ROWS_PER_GATHER)]

        @pl.loop(0, NG)
        def _group(gi):
            goff = gi * (G * 16)
            if first:
                accs = [bias_v[pl.ds(goff + k * 16, 16)] for k in range(G)]
            else:
                accs = [acc[pl.ds(goff + k * 16, 16)] for k in range(G)]
            woff = gi * (GW * 16)
            for j in range(ROWS_PER_GATHER):
                for q in range(GW):
                    x = rbuf[j, pl.ds(woff + q * 16, 16)]
                    accs[2 * q] = accs[2 * q] + lo_f32(x) * vjs[j]
                    accs[2 * q + 1] = accs[2 * q + 1] + hi_f32(x) * vjs[j]
            for k in range(G):
                acc[pl.ds(goff + k * 16, 16)] = accs[k]

        tidx = lane + (LW - 12)
        te = bias_v[pl.ds(L1, 16)] if first else acc[pl.ds(L1, 16)]
        to = bias_v[pl.ds(L1 + 16, 16)] if first else acc[pl.ds(L1 + 16, 16)]
        tse = None
        tso = None
        for j in range(ROWS_PER_GATHER):
            t = plsc.load_gather(rbuf, [full16(j), tidx])
            e = lo_f32(t) * vjs[j]
            o = hi_f32(t) * vjs[j]
            tse = e if tse is None else tse + e
            tso = o if tso is None else tso + o
        acc[pl.ds(L1, 16)] = te + tse * tail_mask
        acc[pl.ds(L1 + 16, 16)] = to + tso * tail_mask

    start_gather(0, 0, 0)
    start_gather(0, 0, 1)

    @pl.loop(0, SPW)
    def _sample(s):
        g = base + s
        s_next = jnp.minimum(s + 1, SPW - 1)

        pltpu.make_async_copy(
            weight.at[idxw_v.at[s, pl.ds(0, ROWS_PER_GATHER)]], rows0, sem0).wait()
        mac(s, 0, 0)
        d_b0 = start_gather(s, 1, 0)

        pltpu.make_async_copy(
            weight.at[idxw_v.at[s, pl.ds(ROWS_PER_GATHER, ROWS_PER_GATHER)]], rows1, sem1).wait()
        mac(s, 0, 1)
        d_b1 = start_gather(s, 1, 1)

        d_b0.wait()
        mac(s, 1, 0)
        start_gather(s_next, 0, 0)

        d_b1.wait()
        mac(s, 1, 1)
        start_gather(s_next, 0, 1)

        usv = plsc.load_gather(us_v, [full16(s)])
        thv = plsc.load_gather(them_v, [full16(s)])
        ftm = ft_v[...]
        inv = 1.0 / ftm
        zero = jnp.zeros((16,), jnp.float32)

        def clampf(x):
            return jnp.minimum(jnp.maximum(x, zero), ftm)

        @pl.loop(0, H // 32)
        def _fin(m):
            o1 = m * 32
            o2 = H + m * 32
            awe1 = acc_w[pl.ds(o1, 16)]
            awo1 = acc_w[pl.ds(o1 + 16, 16)]
            awe2 = acc_w[pl.ds(o2, 16)]
            awo2 = acc_w[pl.ds(o2 + 16, 16)]
            abe1 = acc_b[pl.ds(o1, 16)]
            abo1 = acc_b[pl.ds(o1 + 16, 16)]
            abe2 = acc_b[pl.ds(o2, 16)]
            abo2 = acc_b[pl.ds(o2 + 16, 16)]
            we1 = clampf(usv * awe1 + thv * abe1)
            wo1 = clampf(usv * awo1 + thv * abo1)
            we2 = clampf(usv * awe2 + thv * abe2)
            wo2 = clampf(usv * awo2 + thv * abo2)
            ve1 = clampf(usv * abe1 + thv * awe1)
            vo1 = clampf(usv * abo1 + thv * awo1)
            ve2 = clampf(usv * abe2 + thv * awe2)
            vo2 = clampf(usv * abo2 + thv * awo2)
            ieven = lane * 2 + o1
            iodd = ieven + 1
            plsc.store_scatter(out_stage, [ieven], we1 * we2 * inv)
            plsc.store_scatter(out_stage, [iodd], wo1 * wo2 * inv)
            plsc.store_scatter(out_stage, [ieven + H], ve1 * ve2 * inv)
            plsc.store_scatter(out_stage, [iodd + H], vo1 * vo2 * inv)

        ush = usv - 0.5
        pqe = (acc_w[pl.ds(L1, 16)] - acc_b[pl.ds(L1, 16)]) * ush
        pqo = (acc_w[pl.ds(L1 + 16, 16)] - acc_b[pl.ds(L1 + 16, 16)]) * ush
        pidx = 2 * lane + (L1 - 24)
        pmask = lane >= 12
        plsc.store_scatter(out_stage, [pidx], pqe, mask=pmask)
        plsc.store_scatter(out_stage, [pidx + 1], pqo, mask=pmask)

        pltpu.sync_copy(out_stage, out.at[g])

    pltpu.make_async_copy(
        weight.at[idxw_v.at[SPW - 1, pl.ds(0, ROWS_PER_GATHER)]], rows0, sem0).wait()
    pltpu.make_async_copy(
        weight.at[idxw_v.at[SPW - 1, pl.ds(ROWS_PER_GATHER, ROWS_PER_GATHER)]], rows1, sem1).wait()


@jax.jit
def _run(w_indices, w_values, b_indices, b_values, weight, bias, us, them, ftv):
    mesh = plsc.VectorSubcoreMesh(core_axis_name="c", subcore_axis_name="s",
                                  num_cores=NC, num_subcores=NS)
    f = pl.kernel(
        _sc_kernel,
        out_type=jax.ShapeDtypeStruct((B, D), jnp.float32),
        mesh=mesh,
        scratch_types=[
            pltpu.VMEM((SPW, A), jnp.int32),
            pltpu.VMEM((SPW, A), jnp.int32),
            pltpu.VMEM((SPW, A), jnp.float32),
            pltpu.VMEM((SPW, A), jnp.float32),
            pltpu.VMEM((SPW,), jnp.float32),
            pltpu.VMEM((SPW,), jnp.float32),
            pltpu.VMEM((16,), jnp.float32),
            pltpu.VMEM((ACC,), jnp.float32),
            pltpu.VMEM((ROWS_PER_GATHER, DWP), jnp.int32),
            pltpu.VMEM((ROWS_PER_GATHER, DWP), jnp.int32),
            pltpu.VMEM((ACC,), jnp.float32),
            pltpu.VMEM((ACC,), jnp.float32),
            pltpu.VMEM((D,), jnp.float32),
            pltpu.SemaphoreType.DMA,
            pltpu.SemaphoreType.DMA,
        ],
        compiler_params=pltpu.CompilerParams(use_tc_tiling_on_sc=False,
                                             needs_layout_passes=False),
    )
    return f(w_indices, w_values, b_indices, b_values, us, them, weight, bias, ftv)


def _interleave_bias(bias):
    main = bias[:L1].reshape(L1 // 32, 16, 2)
    main_il = jnp.transpose(main, (0, 2, 1)).reshape(L1)
    tail = bias[L1:].reshape(4, 2)
    te = jnp.pad(tail[:, 0], (12, 0))
    to = jnp.pad(tail[:, 1], (12, 0))
    return jnp.concatenate([main_il, te, to])


def kernel(w_indices, w_values, b_indices, b_values, weight, bias, us, them, ft_max_val):
    ftv = jnp.broadcast_to(jnp.asarray(ft_max_val, jnp.float32), (16,))
    w_bf = weight.astype(jnp.bfloat16)
    w_packed = lax.bitcast_convert_type(w_bf.reshape(weight.shape[0], DW, 2),
                                        jnp.int32)
    w_packed = jnp.pad(w_packed, ((0, 0), (0, DWP - DW)))
    bias_il = _interleave_bias(bias)
    return _run(w_indices.astype(jnp.int32), w_values, b_indices.astype(jnp.int32),
                b_values, w_packed, bias_il, us.reshape(B), them.reshape(B), ftv)


# speedup vs baseline: 2.1773x; 2.1773x over previous
import functools

import jax
import jax.numpy as jnp
from jax import lax
from jax.experimental import pallas as pl
from jax.experimental.pallas import tpu as pltpu
from jax.experimental.pallas import tpu_sc as plsc

L1 = 3072
NPSQT = 8
D = L1 + NPSQT
H = L1 // 2
B = 1024
A = 32
NC = 2
NS = 16
NW = NC * NS
SPW = B // NW
CH = L1 // 16
ROWS_PER_GATHER = 16
ACC = L1 + 16


def _sc_kernel(w_idx, w_val, b_idx, b_val, us, them, weight, bias, ftv, out,
               idxw_v, idxb_v, vw_v, vb_v, us_v, them_v, ft_v, bias_v,
               rows0, rows1, acc_w, acc_b, out_stage, sem0, sem1):
    wid = lax.axis_index("s") * NC + lax.axis_index("c")
    base = wid * SPW

    pltpu.sync_copy(w_idx.at[pl.ds(base, SPW)], idxw_v)
    pltpu.sync_copy(b_idx.at[pl.ds(base, SPW)], idxb_v)
    pltpu.sync_copy(w_val.at[pl.ds(base, SPW)], vw_v)
    pltpu.sync_copy(b_val.at[pl.ds(base, SPW)], vb_v)
    pltpu.sync_copy(us.at[pl.ds(base, SPW)], us_v)
    pltpu.sync_copy(them.at[pl.ds(base, SPW)], them_v)
    pltpu.sync_copy(bias, bias_v)
    pltpu.sync_copy(ftv, ft_v)

    lane = lax.broadcasted_iota(jnp.int32, (16,), 0)
    tail_mask = (lane >= 8).astype(jnp.float32)
    rows = (rows0, rows1)
    sems = (sem0, sem1)
    idxs = (idxw_v, idxb_v)
    vals = (vw_v, vb_v)

    def start_gather(s, p, h):
        src = weight.at[idxs[p].at[s, pl.ds(h * ROWS_PER_GATHER, ROWS_PER_GATHER)]]
        return pltpu.async_copy(src, rows[h], sems[h])

    def full16(v):
        return jnp.full((16,), v, dtype=jnp.int32)

    G = 8
    NG = CH // G

    def mac(s, p, h):
        acc = (acc_w, acc_b)[p]
        rbuf = rows[h]
        vref = vals[p]
        first = (h == 0)

        vjs = [plsc.load_gather(vref, [full16(s), full16(h * ROWS_PER_GATHER + j)])
               for j in range(ROWS_PER_GATHER)]

        @pl.loop(0, NG)
        def _group(gi):
            goff = gi * (G * 16)
            if first:
                accs = [bias_v[pl.ds(goff + k * 16, 16)] for k in range(G)]
            else:
                accs = [acc[pl.ds(goff + k * 16, 16)] for k in range(G)]
            for j in range(ROWS_PER_GATHER):
                for k in range(G):
                    accs[k] = accs[k] + rbuf[j, pl.ds(goff + k * 16, 16)] * vjs[j]
            for k in range(G):
                acc[pl.ds(goff + k * 16, 16)] = accs[k]

        tacc = bias_v[pl.ds(D - 16, 16)] if first else acc[pl.ds(L1, 16)]
        tsum = None
        for j in range(ROWS_PER_GATHER):
            t = plsc.load_gather(rbuf, [full16(j), lane + (D - 16)]) * vjs[j]
            tsum = t if tsum is None else tsum + t
        acc[pl.ds(L1, 16)] = tacc + tsum * tail_mask if not first else (
            tacc * tail_mask + tsum * tail_mask)

    start_gather(0, 0, 0)
    start_gather(0, 0, 1)

    @pl.loop(0, SPW)
    def _sample(s):
        g = base + s
        s_next = jnp.minimum(s + 1, SPW - 1)

        pltpu.make_async_copy(
            weight.at[idxw_v.at[s, pl.ds(0, ROWS_PER_GATHER)]], rows0, sem0).wait()
        mac(s, 0, 0)
        d_b0 = start_gather(s, 1, 0)

        pltpu.make_async_copy(
            weight.at[idxw_v.at[s, pl.ds(ROWS_PER_GATHER, ROWS_PER_GATHER)]], rows1, sem1).wait()
        mac(s, 0, 1)
        d_b1 = start_gather(s, 1, 1)

        d_b0.wait()
        mac(s, 1, 0)
        start_gather(s_next, 0, 0)

        d_b1.wait()
        mac(s, 1, 1)
        start_gather(s_next, 0, 1)

        usv = plsc.load_gather(us_v, [full16(s)])
        thv = plsc.load_gather(them_v, [full16(s)])
        ftm = ft_v[...]
        inv = 1.0 / ftm
        zero = jnp.zeros((16,), jnp.float32)

        def clampf(x):
            return jnp.minimum(jnp.maximum(x, zero), ftm)

        @pl.loop(0, H // 16, unroll=2)
        def _fin(c):
            o1 = c * 16
            o2 = H + c * 16
            aw1 = acc_w[pl.ds(o1, 16)]
            aw2 = acc_w[pl.ds(o2, 16)]
            ab1 = acc_b[pl.ds(o1, 16)]
            ab2 = acc_b[pl.ds(o2, 16)]
            w1 = clampf(usv * aw1 + thv * ab1)
            w2 = clampf(usv * aw2 + thv * ab2)
            v1 = clampf(usv * ab1 + thv * aw1)
            v2 = clampf(usv * ab2 + thv * aw2)
            out_stage[pl.ds(o1, 16)] = w1 * w2 * inv
            out_stage[pl.ds(H + o1, 16)] = v1 * v2 * inv

        pq = (acc_w[pl.ds(L1, 16)] - acc_b[pl.ds(L1, 16)]) * (usv - 0.5)
        plsc.store_scatter(out_stage, [lane + (D - 16)], pq, mask=lane >= 8)

        pltpu.sync_copy(out_stage, out.at[g])

    pltpu.make_async_copy(
        weight.at[idxw_v.at[SPW - 1, pl.ds(0, ROWS_PER_GATHER)]], rows0, sem0).wait()
    pltpu.make_async_copy(
        weight.at[idxw_v.at[SPW - 1, pl.ds(ROWS_PER_GATHER, ROWS_PER_GATHER)]], rows1, sem1).wait()


@jax.jit
def _run(w_indices, w_values, b_indices, b_values, weight, bias, us, them, ftv):
    mesh = plsc.VectorSubcoreMesh(core_axis_name="c", subcore_axis_name="s")
    f = pl.kernel(
        _sc_kernel,
        out_type=jax.ShapeDtypeStruct((B, D), jnp.float32),
        mesh=mesh,
        scratch_types=[
            pltpu.VMEM((SPW, A), jnp.int32),
            pltpu.VMEM((SPW, A), jnp.int32),
            pltpu.VMEM((SPW, A), jnp.float32),
            pltpu.VMEM((SPW, A), jnp.float32),
            pltpu.VMEM((SPW,), jnp.float32),
            pltpu.VMEM((SPW,), jnp.float32),
            pltpu.VMEM((16,), jnp.float32),
            pltpu.VMEM((D,), jnp.float32),
            pltpu.VMEM((ROWS_PER_GATHER, D), jnp.float32),
            pltpu.VMEM((ROWS_PER_GATHER, D), jnp.float32),
            pltpu.VMEM((ACC,), jnp.float32),
            pltpu.VMEM((ACC,), jnp.float32),
            pltpu.VMEM((D,), jnp.float32),
            pltpu.SemaphoreType.DMA,
            pltpu.SemaphoreType.DMA,
        ],
        compiler_params=pltpu.CompilerParams(use_tc_tiling_on_sc=False,
                                             needs_layout_passes=False),
    )
    return f(w_indices, w_values, b_indices, b_values, us, them, weight, bias, ftv)


def kernel(w_indices, w_values, b_indices, b_values, weight, bias, us, them, ft_max_val):
    ftv = jnp.broadcast_to(jnp.asarray(ft_max_val, jnp.float32), (16,))
    w_flat = jax.lax.optimization_barrier(weight.reshape(-1))
    w_lin = w_flat.reshape(weight.shape)
    return _run(w_indices.astype(jnp.int32), w_values, b_indices.astype(jnp.int32),
                b_values, w_lin, bias, us.reshape(B), them.reshape(B), ftv)
